# baseline (speedup 1.0000x reference)
import functools
import jax
import jax.numpy as jnp
from jax import lax
from jax.experimental import pallas as pl
from jax.experimental.pallas import tpu as pltpu
from jax.experimental.pallas import tpu_sc as plsc

VOCAB = 1000000
B = 16384
K = 20
D = 64
W = D // 2
NW = 32
BW = B // NW
C = 32
NSTEP = BW // C
IDXROW = 128
XROWS_PER_CHUNK = (C * K) // IDXROW
TBLK = 4096


def _pack_words(x):
    u = lax.bitcast_convert_type(x, jnp.uint32) + 0x8000
    lo = u[:W] >> 16
    hi = u[W:] & jnp.uint32(0xFFFF0000)
    return lax.bitcast_convert_type(lo | hi, jnp.float32)


def _pack_kernel(ct_ref, xt_ref, o_ref):
    h = TBLK // 2
    cw = _pack_words(ct_ref[...])
    xw = _pack_words(xt_ref[...])
    z = jnp.concatenate([cw[:, :h], xw[:, :h], cw[:, h:], xw[:, h:]], axis=0)
    o_ref[...] = z.T


def _sc_kernel(cl_hbm, xl_hbm, tab_hbm, out_hbm,
               cidx, xidx, crow, xrow, obuf, sbuf, sem_a, sem_b):
    nc = 2
    wid = lax.axis_index("s") * nc + lax.axis_index("c")
    ck = C * K

    iota = lax.iota(jnp.int32, 16)
    hi_mask = iota < 4

    def unpk(v):
        return plsc.unpack(plsc.bitcast(v, jnp.bfloat16),
                           format=plsc.PackFormat.INTERLEAVED)

    pltpu.sync_copy(cl_hbm.at[wid], cidx)
    pltpu.sync_copy(xl_hbm.at[wid], xidx)

    def fire(s, boff, sem):
        pltpu.async_copy(tab_hbm.at[cidx.at[pl.ds(s * C, C)]],
                         crow.at[pl.ds(boff * C, C)], sem)
        for j in range(XROWS_PER_CHUNK):
            pltpu.async_copy(tab_hbm.at[xidx.at[s * XROWS_PER_CHUNK + j]],
                             xrow.at[pl.ds(boff * ck + j * IDXROW, IDXROW)],
                             sem)

    def wait_chunk(boff, sem):
        pltpu.make_async_copy(tab_hbm.at[pl.ds(0, C)],
                              crow.at[pl.ds(boff * C, C)], sem).wait()
        for j in range(XROWS_PER_CHUNK):
            pltpu.make_async_copy(
                tab_hbm.at[pl.ds(0, IDXROW)],
                xrow.at[pl.ds(boff * ck + j * IDXROW, IDXROW)], sem).wait()

    def compute(s, boff):
        def per_group(g, _):
            flat = g * 16 + iota
            xrows = boff * ck + flat
            crows = boff * C + flat // K
            acc = None
            for w in range(W):
                cw = plsc.load_gather(crow, [crows, jnp.full((16,), w, jnp.int32)])
                xw = plsc.load_gather(xrow, [xrows, jnp.full((16,), W + w, jnp.int32)])
                ca, cb = unpk(cw)
                a, b = unpk(xw)
                t = ca * a + cb * b
                acc = t if acc is None else acc + t
            obuf[pl.ds(g * 16, 16)] = acc
            return 0

        lax.fori_loop(0, ck // 16, per_group, 0)
        pltpu.sync_copy(obuf, out_hbm.at[pl.ds(wid * BW * K + s * ck, ck)])

    fire(0, 0, sem_a)

    def dstep(t, _):
        s0 = 2 * t
        fire(s0 + 1, 1, sem_b)
        wait_chunk(0, sem_a)
        compute(s0, 0)

        @pl.when(s0 + 2 < NSTEP)
        def _():
            fire(s0 + 2, 0, sem_a)

        wait_chunk(1, sem_b)
        compute(s0 + 1, 1)
        return 0

    lax.fori_loop(0, NSTEP // 2, dstep, 0)


@jax.jit
def kernel(center_labels, context_labels, center_table, context_table):
    nblk = (VOCAB + TBLK - 1) // TBLK
    packed = pl.pallas_call(
        _pack_kernel,
        grid=(nblk,),
        in_specs=[
            pl.BlockSpec((D, TBLK), lambda i: (0, i)),
            pl.BlockSpec((D, TBLK), lambda i: (0, i)),
        ],
        out_specs=pl.BlockSpec((TBLK // 2, 4 * W), lambda i: (i, 0)),
        out_shape=jax.ShapeDtypeStruct((nblk * (TBLK // 2), 4 * W), jnp.float32),
        compiler_params=pltpu.CompilerParams(fuse_transposed_lhs_in_matmul=True),
    )(center_table.T, context_table.T)
    tab = packed.reshape(nblk * TBLK, D)

    def remap(v):
        ib = v // TBLK
        l = v % TBLK
        half = (l >= TBLK // 2).astype(jnp.int32)
        ql = l - half * (TBLK // 2)
        return (ib * (TBLK // 2) + ql) * 2 + half

    mesh = plsc.VectorSubcoreMesh(core_axis_name="c", subcore_axis_name="s")
    k = functools.partial(
        pl.kernel,
        out_type=jax.ShapeDtypeStruct((B * K,), jnp.float32),
        mesh=mesh,
        compiler_params=pltpu.CompilerParams(needs_layout_passes=False,
                                             use_tc_tiling_on_sc=False),
        scratch_types=[
            pltpu.VMEM((BW,), jnp.int32),
            pltpu.VMEM((BW * K // IDXROW, IDXROW), jnp.int32),
            pltpu.VMEM((2 * C, D), jnp.float32),
            pltpu.VMEM((2 * C * K, D), jnp.float32),
            pltpu.VMEM((C * K,), jnp.float32),
            pltpu.VMEM((16 * 32,), jnp.float32),
            pltpu.SemaphoreType.DMA,
            pltpu.SemaphoreType.DMA,
        ],
    )(_sc_kernel)
    out = k(remap(center_labels).reshape(NW, BW),
            remap(context_labels).reshape(NW, BW * K // IDXROW, IDXROW),
            tab)
    return out.reshape(B, K)


# speedup vs baseline: 1.3710x; 1.3710x over previous
import functools
import jax
import jax.numpy as jnp
from jax import lax
from jax.experimental import pallas as pl
from jax.experimental.pallas import tpu as pltpu
from jax.experimental.pallas import tpu_sc as plsc

VOCAB = 1000000
B = 16384
K = 20
D = 64
W = D // 2
NW = 32
BW = B // NW
C = 32
NSTEP = BW // C
IDXROW = 128
XROWS_PER_CHUNK = (C * K) // IDXROW
TBLK = 8192


def _pack_words(x):
    u = lax.bitcast_convert_type(x, jnp.uint32) + 0x8000
    lo = u[:W] >> 16
    hi = u[W:] & jnp.uint32(0xFFFF0000)
    return lax.bitcast_convert_type(lo | hi, jnp.float32)


def _pack_kernel(ct_ref, xt_ref, o_ref):
    h = TBLK // 2
    cw = _pack_words(ct_ref[...])
    xw = _pack_words(xt_ref[...])
    z = jnp.concatenate([cw[:, :h], xw[:, :h], cw[:, h:], xw[:, h:]], axis=0)
    o_ref[...] = z.T


def _sc_kernel(cl_hbm, xl_hbm, tab_hbm, out_hbm,
               cidx, xidx, crow, xrow, obuf, sbuf, sem_a, sem_b):
    nc = 2
    wid = lax.axis_index("s") * nc + lax.axis_index("c")
    ck = C * K

    iota = lax.iota(jnp.int32, 16)
    hi_mask = iota < 4

    def unpk(v):
        return plsc.unpack(plsc.bitcast(v, jnp.bfloat16),
                           format=plsc.PackFormat.INTERLEAVED)

    pltpu.sync_copy(cl_hbm.at[wid], cidx)
    pltpu.sync_copy(xl_hbm.at[wid], xidx)

    def fire(s, boff, sem):
        pltpu.async_copy(tab_hbm.at[cidx.at[pl.ds(s * C, C)]],
                         crow.at[pl.ds(boff * C, C)], sem)
        for j in range(XROWS_PER_CHUNK):
            pltpu.async_copy(tab_hbm.at[xidx.at[s * XROWS_PER_CHUNK + j]],
                             xrow.at[pl.ds(boff * ck + j * IDXROW, IDXROW)],
                             sem)

    def wait_chunk(boff, sem):
        pltpu.make_async_copy(tab_hbm.at[pl.ds(0, C)],
                              crow.at[pl.ds(boff * C, C)], sem).wait()
        for j in range(XROWS_PER_CHUNK):
            pltpu.make_async_copy(
                tab_hbm.at[pl.ds(0, IDXROW)],
                xrow.at[pl.ds(boff * ck + j * IDXROW, IDXROW)], sem).wait()

    def compute(s, boff):
        def per_center(b, _):
            ca1, cb1 = unpk(crow[boff * C + b, pl.ds(0, 16)])
            ca2, cb2 = unpk(crow[boff * C + b, pl.ds(16, 16)])
            for k in range(K):
                r = boff * ck + b * K + k
                a1, b1 = unpk(xrow[r, pl.ds(0, 16)])
                a2, b2 = unpk(xrow[r, pl.ds(16, 16)])
                p = ca1 * a1
                p = p + cb1 * b1
                p = p + ca2 * a2
                p = p + cb2 * b2
                plsc.store_scatter(sbuf, [iota * 32 + k], p)
            s_lo = sbuf[pl.ds(0, 16)]
            s_hi = sbuf[pl.ds(16, 16)]
            for l in range(1, 16):
                s_lo = s_lo + sbuf[pl.ds(l * 32, 16)]
                s_hi = s_hi + sbuf[pl.ds(l * 32 + 16, 16)]
            plsc.store_scatter(obuf, [b * K + iota], s_lo)
            plsc.store_scatter(obuf, [jnp.minimum(b * K + 16 + iota, ck - 1)],
                               s_hi, mask=hi_mask)
            return 0

        lax.fori_loop(0, C, per_center, 0)
        pltpu.sync_copy(obuf, out_hbm.at[pl.ds(wid * BW * K + s * ck, ck)])

    fire(0, 0, sem_a)

    def dstep(t, _):
        s0 = 2 * t
        fire(s0 + 1, 1, sem_b)
        wait_chunk(0, sem_a)
        compute(s0, 0)

        @pl.when(s0 + 2 < NSTEP)
        def _():
            fire(s0 + 2, 0, sem_a)

        wait_chunk(1, sem_b)
        compute(s0 + 1, 1)
        return 0

    lax.fori_loop(0, NSTEP // 2, dstep, 0)


@jax.jit
def kernel(center_labels, context_labels, center_table, context_table):
    nblk = (VOCAB + TBLK - 1) // TBLK
    packed = pl.pallas_call(
        _pack_kernel,
        grid=(nblk,),
        in_specs=[
            pl.BlockSpec((D, TBLK), lambda i: (0, i)),
            pl.BlockSpec((D, TBLK), lambda i: (0, i)),
        ],
        out_specs=pl.BlockSpec((TBLK // 2, 4 * W), lambda i: (i, 0)),
        out_shape=jax.ShapeDtypeStruct((nblk * (TBLK // 2), 4 * W), jnp.float32),
        compiler_params=pltpu.CompilerParams(fuse_transposed_lhs_in_matmul=True),
    )(center_table.T, context_table.T)
    tab = packed.reshape(nblk * TBLK * 2, W)

    def remap(v, ctx):
        ib = v // TBLK
        l = v % TBLK
        half = (l >= TBLK // 2).astype(jnp.int32)
        ql = l - half * (TBLK // 2)
        return ((ib * (TBLK // 2) + ql) * 2 + half) * 2 + ctx

    mesh = plsc.VectorSubcoreMesh(core_axis_name="c", subcore_axis_name="s")
    k = functools.partial(
        pl.kernel,
        out_type=jax.ShapeDtypeStruct((B * K,), jnp.float32),
        mesh=mesh,
        compiler_params=pltpu.CompilerParams(needs_layout_passes=False,
                                             use_tc_tiling_on_sc=False),
        scratch_types=[
            pltpu.VMEM((BW,), jnp.int32),
            pltpu.VMEM((BW * K // IDXROW, IDXROW), jnp.int32),
            pltpu.VMEM((2 * C, W), jnp.float32),
            pltpu.VMEM((2 * C * K, W), jnp.float32),
            pltpu.VMEM((C * K,), jnp.float32),
            pltpu.VMEM((16 * 32,), jnp.float32),
            pltpu.SemaphoreType.DMA,
            pltpu.SemaphoreType.DMA,
        ],
    )(_sc_kernel)
    out = k(remap(center_labels, 0).reshape(NW, BW),
            remap(context_labels, 1).reshape(NW, BW * K // IDXROW, IDXROW),
            tab)
    return out.reshape(B, K)
